# baseline (speedup 1.0000x reference)
import functools

import jax
import jax.numpy as jnp
from jax import lax
from jax.experimental import pallas as pl
from jax.experimental.pallas import tpu as pltpu
from jax.experimental.pallas import tpu_sc as plsc

_L = 16
_NS = 16
_NC = 2


def _sc_body(n, d, feat_hbm, out_hbm, x_v, f0_v, rd_v, g_v, red_v, part_v, all_v, part_sh):
    c = lax.axis_index("c")
    s = lax.axis_index("s")
    rows = n // _NS
    base = s * rows

    pltpu.sync_copy(feat_hbm.at[pl.ds((c * n + base) * d, rows * d)], x_v)
    pltpu.sync_copy(feat_hbm.at[pl.ds(c * n * d, d)], f0_v)

    for h in range(d // _L):
        v = jnp.maximum(f0_v[pl.ds(h * _L, _L)], 0.0)
        rd_v[pl.ds(h * _L, _L)] = 1.0 / jnp.exp(v)

    nb = rows // _L
    lane = lax.iota(jnp.int32, _L)
    rd0 = rd_v[pl.ds(0, _L)]
    rd1 = rd_v[pl.ds(_L, _L)]

    zeros = jnp.zeros((_L,), jnp.float32)
    for r in range(4 * _L):
        red_v[pl.ds(r * 2 * _L + _L, _L)] = zeros

    def _hmax(t, rb):
        red_v[pl.ds(rb, _L)] = t
        x = t
        for k in (8, 4, 2, 1):
            y = red_v[pl.ds(rb + k, _L)]
            x = jnp.maximum(x, y)
            if k != 1:
                red_v[pl.ds(rb, _L)] = x
        return x[0]

    def block(b, ss, roff):
        mvec = jnp.zeros((_L,), jnp.float32)
        qvec = jnp.zeros((_L,), jnp.float32)
        for r in range(_L):
            off = b * (_L * d) + r * d
            v0 = x_v[pl.ds(off, _L)]
            v1 = x_v[pl.ds(off + _L, _L)]
            fa = jnp.maximum(v0, 0.0)
            fb = jnp.maximum(v1, 0.0)
            qa = (jnp.exp(fa) * rd0) * fa
            qb = (jnp.exp(fb) * rd1) * fb
            mr = _hmax(jnp.maximum(fa, fb), roff + r * 2 * _L)
            qr = _hmax(jnp.maximum(qa, qb), roff + (_L + r) * 2 * _L)
            mvec = jnp.where(lane == r, mr, mvec)
            qvec = jnp.where(lane == r, qr, qvec)
        gvec = qvec * (1.0 / mvec)
        g_v[pl.ds(b * _L, _L)] = gvec
        return ss + gvec * gvec

    def block2(b2, ss):
        ss = block(2 * b2, ss, 0)
        return block(2 * b2 + 1, ss, 2 * _L * 2 * _L)

    ss = lax.fori_loop(0, nb // 2, block2, jnp.zeros((_L,), jnp.float32))

    part_v[...] = ss
    pltpu.sync_copy(part_v, part_sh.at[pl.ds(s * _L, _L)])
    plsc.subcore_barrier()
    pltpu.sync_copy(part_sh, all_v)
    tv = all_v[pl.ds(0, _L)]
    for i in range(1, _NS):
        tv = tv + all_v[pl.ds(i * _L, _L)]
    tot = tv[0]
    for l in range(1, _L):
        tot = tot + tv[l]

    tv = jnp.full((_L,), tot, jnp.float32)
    y = 0.5 * (1.0 + tv)
    for _ in range(32):
        y = 0.5 * (y + tv / y)
    r = 1.0 / y

    for b in range(nb):
        g_v[pl.ds(b * _L, _L)] = g_v[pl.ds(b * _L, _L)] * r
    pltpu.sync_copy(g_v, out_hbm.at[pl.ds(c * n + base, rows)])


def kernel(coords, features, len_batch):
    b, n, d = features.shape
    mesh = plsc.VectorSubcoreMesh(
        core_axis_name="c", subcore_axis_name="s", num_cores=_NC, num_subcores=_NS
    )
    rows = n // _NS
    run = pl.kernel(
        functools.partial(_sc_body, n, d),
        out_type=jax.ShapeDtypeStruct((b * n,), features.dtype),
        mesh=mesh,
        scratch_types=[
            pltpu.VMEM((rows * d,), jnp.float32),
            pltpu.VMEM((d,), jnp.float32),
            pltpu.VMEM((d,), jnp.float32),
            pltpu.VMEM((rows,), jnp.float32),
            pltpu.VMEM((4 * _L * 2 * _L,), jnp.float32),
            pltpu.VMEM((_L,), jnp.float32),
            pltpu.VMEM((_NS * _L,), jnp.float32),
            pltpu.VMEM_SHARED((_NS * _L,), jnp.float32),
        ],
    )
    return run(features.reshape(b * n * d))


# speedup vs baseline: 1.0245x; 1.0245x over previous
import functools

import jax
import jax.numpy as jnp
from jax import lax
from jax.experimental import pallas as pl
from jax.experimental.pallas import tpu as pltpu
from jax.experimental.pallas import tpu_sc as plsc

_L = 16
_NS = 16
_NC = 2


def _sc_body(n, d, feat_hbm, out_hbm, x_v, f0_v, rd_v, g_v, part_v, all_v, part_sh, *red_v):
    c = lax.axis_index("c")
    s = lax.axis_index("s")
    rows = n // _NS
    base = s * rows

    pltpu.sync_copy(feat_hbm.at[pl.ds((c * n + base) * d, rows * d)], x_v)
    pltpu.sync_copy(feat_hbm.at[pl.ds(c * n * d, d)], f0_v)

    for h in range(d // _L):
        v = jnp.maximum(f0_v[pl.ds(h * _L, _L)], 0.0)
        rd_v[pl.ds(h * _L, _L)] = 1.0 / jnp.exp(v)

    nb = rows // _L
    lane = lax.iota(jnp.int32, _L)
    rd0 = rd_v[pl.ds(0, _L)]
    rd1 = rd_v[pl.ds(_L, _L)]

    zeros = jnp.zeros((_L,), jnp.float32)
    for r in range(_L):
        red_v[r][pl.ds(_L, _L)] = zeros
        red_v[r][pl.ds(3 * _L, _L)] = zeros

    def _hmax(t, ref, rb):
        ref[pl.ds(rb, _L)] = t
        x = t
        for k in (8, 4, 2, 1):
            y = ref[pl.ds(rb + k, _L)]
            x = jnp.maximum(x, y)
            if k != 1:
                ref[pl.ds(rb, _L)] = x
        return x[0]

    def block(b, ss):
        mvec = jnp.zeros((_L,), jnp.float32)
        qvec = jnp.zeros((_L,), jnp.float32)
        for r in range(_L):
            off = b * (_L * d) + r * d
            v0 = x_v[pl.ds(off, _L)]
            v1 = x_v[pl.ds(off + _L, _L)]
            fa = jnp.maximum(v0, 0.0)
            fb = jnp.maximum(v1, 0.0)
            qa = (jnp.exp(fa) * rd0) * fa
            qb = (jnp.exp(fb) * rd1) * fb
            mr = _hmax(jnp.maximum(fa, fb), red_v[r], 0)
            qr = _hmax(jnp.maximum(qa, qb), red_v[r], 2 * _L)
            mvec = jnp.where(lane == r, mr, mvec)
            qvec = jnp.where(lane == r, qr, qvec)
        gvec = qvec * (1.0 / mvec)
        g_v[pl.ds(b * _L, _L)] = gvec
        return ss + gvec * gvec

    ss = lax.fori_loop(0, nb, block, jnp.zeros((_L,), jnp.float32))

    part_v[...] = ss
    pltpu.sync_copy(part_v, part_sh.at[pl.ds(s * _L, _L)])
    plsc.subcore_barrier()
    pltpu.sync_copy(part_sh, all_v)
    tv = all_v[pl.ds(0, _L)]
    for i in range(1, _NS):
        tv = tv + all_v[pl.ds(i * _L, _L)]
    tot = tv[0]
    for l in range(1, _L):
        tot = tot + tv[l]

    tv = jnp.full((_L,), tot, jnp.float32)
    y = 0.5 * (1.0 + tv)
    for _ in range(24):
        y = 0.5 * (y + tv / y)
    r = 1.0 / y

    for b in range(nb):
        g_v[pl.ds(b * _L, _L)] = g_v[pl.ds(b * _L, _L)] * r
    pltpu.sync_copy(g_v, out_hbm.at[pl.ds(c * n + base, rows)])


def kernel(coords, features, len_batch):
    b, n, d = features.shape
    mesh = plsc.VectorSubcoreMesh(
        core_axis_name="c", subcore_axis_name="s", num_cores=_NC, num_subcores=_NS
    )
    rows = n // _NS
    run = pl.kernel(
        functools.partial(_sc_body, n, d),
        out_type=jax.ShapeDtypeStruct((b * n,), features.dtype),
        mesh=mesh,
        scratch_types=[
            pltpu.VMEM((rows * d,), jnp.float32),
            pltpu.VMEM((d,), jnp.float32),
            pltpu.VMEM((d,), jnp.float32),
            pltpu.VMEM((rows,), jnp.float32),
            pltpu.VMEM((_L,), jnp.float32),
            pltpu.VMEM((_NS * _L,), jnp.float32),
            pltpu.VMEM_SHARED((_NS * _L,), jnp.float32),
        ] + [pltpu.VMEM((4 * _L,), jnp.float32) for _ in range(_L)],
    )
    return run(features.reshape(b * n * d))


# speedup vs baseline: 1.2385x; 1.2089x over previous
import functools

import jax
import jax.numpy as jnp
from jax import lax
from jax.experimental import pallas as pl
from jax.experimental.pallas import tpu as pltpu
from jax.experimental.pallas import tpu_sc as plsc

_L = 16
_NS = 16
_NC = 2


def _sc_body(n, d, feat_hbm, out_hbm, x_v, f0_v, rd_v, g_v, part_v, all_v, part_sh, *red_v):
    c = lax.axis_index("c")
    s = lax.axis_index("s")
    rows = n // _NS
    base = s * rows

    pltpu.sync_copy(feat_hbm.at[pl.ds((c * n + base) * d, rows * d)], x_v)
    pltpu.sync_copy(feat_hbm.at[pl.ds(c * n * d, d)], f0_v)

    for h in range(d // _L):
        rd_v[pl.ds(h * _L, _L)] = jnp.maximum(f0_v[pl.ds(h * _L, _L)], 0.0)

    nb = rows // _L
    lane = lax.iota(jnp.int32, _L)
    f0a = rd_v[pl.ds(0, _L)]
    f0b = rd_v[pl.ds(_L, _L)]

    zeros = jnp.zeros((_L,), jnp.float32)
    for r in range(_L):
        red_v[r][pl.ds(_L, _L)] = zeros
        red_v[r][pl.ds(3 * _L, _L)] = zeros

    def block(b, ss):
        mvec = jnp.zeros((_L,), jnp.float32)
        qvec = jnp.zeros((_L,), jnp.float32)
        for g0 in range(0, _L, 8):
            slots = []
            for r in range(g0, g0 + 8):
                off = b * (_L * d) + r * d
                v0 = x_v[pl.ds(off, _L)]
                v1 = x_v[pl.ds(off + _L, _L)]
                fa = jnp.maximum(v0, 0.0)
                fb = jnp.maximum(v1, 0.0)
                qa = jnp.exp(fa - f0a) * fa
                qb = jnp.exp(fb - f0b) * fb
                slots.append((jnp.maximum(fa, fb), red_v[r], 0))
                slots.append((jnp.maximum(qa, qb), red_v[r], 2 * _L))
            xs = [t for (t, _, _) in slots]
            for x, (_, ref, base) in zip(xs, slots):
                ref[pl.ds(base, _L)] = x
            for k in (8, 4, 2, 1):
                ys = [ref[pl.ds(base + k, _L)] for (_, ref, base) in slots]
                xs = [jnp.maximum(x, y) for x, y in zip(xs, ys)]
                if k != 1:
                    for x, (_, ref, base) in zip(xs, slots):
                        ref[pl.ds(base, _L)] = x
            for i, r in enumerate(range(g0, g0 + 8)):
                mvec = jnp.where(lane == r, xs[2 * i][0], mvec)
                qvec = jnp.where(lane == r, xs[2 * i + 1][0], qvec)
        gvec = qvec * (1.0 / mvec)
        g_v[pl.ds(b * _L, _L)] = gvec
        return ss + gvec * gvec

    ss = lax.fori_loop(0, nb, block, jnp.zeros((_L,), jnp.float32))

    part_v[...] = ss
    pltpu.sync_copy(part_v, part_sh.at[pl.ds(s * _L, _L)])
    plsc.subcore_barrier()
    pltpu.sync_copy(part_sh, all_v)
    tv = all_v[pl.ds(0, _L)]
    for i in range(1, _NS):
        tv = tv + all_v[pl.ds(i * _L, _L)]
    tot = tv[0]
    for l in range(1, _L):
        tot = tot + tv[l]

    tv = jnp.full((_L,), tot, jnp.float32)
    y = 0.5 * (1.0 + tv)
    for _ in range(24):
        y = 0.5 * (y + tv / y)
    r = 1.0 / y

    for b in range(nb):
        g_v[pl.ds(b * _L, _L)] = g_v[pl.ds(b * _L, _L)] * r
    pltpu.sync_copy(g_v, out_hbm.at[pl.ds(c * n + base, rows)])


def kernel(coords, features, len_batch):
    b, n, d = features.shape
    mesh = plsc.VectorSubcoreMesh(
        core_axis_name="c", subcore_axis_name="s", num_cores=_NC, num_subcores=_NS
    )
    rows = n // _NS
    run = pl.kernel(
        functools.partial(_sc_body, n, d),
        out_type=jax.ShapeDtypeStruct((b * n,), features.dtype),
        mesh=mesh,
        scratch_types=[
            pltpu.VMEM((rows * d,), jnp.float32),
            pltpu.VMEM((d,), jnp.float32),
            pltpu.VMEM((d,), jnp.float32),
            pltpu.VMEM((rows,), jnp.float32),
            pltpu.VMEM((_L,), jnp.float32),
            pltpu.VMEM((_NS * _L,), jnp.float32),
            pltpu.VMEM_SHARED((_NS * _L,), jnp.float32),
        ] + [pltpu.VMEM((4 * _L,), jnp.float32) for _ in range(_L)],
    )
    return run(features.reshape(b * n * d))


# speedup vs baseline: 1.2599x; 1.0173x over previous
import functools

import jax
import jax.numpy as jnp
from jax import lax
from jax.experimental import pallas as pl
from jax.experimental.pallas import tpu as pltpu
from jax.experimental.pallas import tpu_sc as plsc

_L = 16
_NS = 16
_NC = 2


def _sc_body(n, d, feat_hbm, out_hbm, x_v, f0_v, rd_v, g_v, part_v, all_v, part_sh, dma_sem, *red_v):
    c = lax.axis_index("c")
    s = lax.axis_index("s")
    rows = n // _NS
    base = s * rows

    cp = pltpu.async_copy(feat_hbm.at[pl.ds((c * n + base) * d, rows * d)], x_v, dma_sem)
    pltpu.sync_copy(feat_hbm.at[pl.ds(c * n * d, d)], f0_v)

    for h in range(d // _L):
        rd_v[pl.ds(h * _L, _L)] = jnp.maximum(f0_v[pl.ds(h * _L, _L)], 0.0)

    nb = rows // _L
    lane = lax.iota(jnp.int32, _L)
    f0a = rd_v[pl.ds(0, _L)]
    f0b = rd_v[pl.ds(_L, _L)]

    zeros = jnp.zeros((_L,), jnp.float32)
    for r in range(_L):
        red_v[r][pl.ds(_L, _L)] = zeros
        red_v[r][pl.ds(3 * _L, _L)] = zeros

    def block(b, ss):
        mvec = jnp.zeros((_L,), jnp.float32)
        qvec = jnp.zeros((_L,), jnp.float32)
        for g0 in range(0, _L, 8):
            slots = []
            for r in range(g0, g0 + 8):
                off = b * (_L * d) + r * d
                v0 = x_v[pl.ds(off, _L)]
                v1 = x_v[pl.ds(off + _L, _L)]
                fa = jnp.maximum(v0, 0.0)
                fb = jnp.maximum(v1, 0.0)
                qa = jnp.exp(fa - f0a) * fa
                qb = jnp.exp(fb - f0b) * fb
                slots.append((jnp.maximum(fa, fb), red_v[r], 0))
                slots.append((jnp.maximum(qa, qb), red_v[r], 2 * _L))
            xs = [t for (t, _, _) in slots]
            for x, (_, ref, base) in zip(xs, slots):
                ref[pl.ds(base, _L)] = x
            for k in (8, 4, 2, 1):
                ys = [ref[pl.ds(base + k, _L)] for (_, ref, base) in slots]
                xs = [jnp.maximum(x, y) for x, y in zip(xs, ys)]
                if k != 1:
                    for x, (_, ref, base) in zip(xs, slots):
                        ref[pl.ds(base, _L)] = x
            for i, r in enumerate(range(g0, g0 + 8)):
                mvec = jnp.where(lane == r, xs[2 * i][0], mvec)
                qvec = jnp.where(lane == r, xs[2 * i + 1][0], qvec)
        gvec = qvec * (1.0 / mvec)
        g_v[pl.ds(b * _L, _L)] = gvec
        return ss + gvec * gvec

    cp.wait()
    ss = lax.fori_loop(0, nb, block, jnp.zeros((_L,), jnp.float32))

    part_v[...] = ss
    pltpu.sync_copy(part_v, part_sh.at[pl.ds(s * _L, _L)])
    plsc.subcore_barrier()
    pltpu.sync_copy(part_sh, all_v)
    tv = all_v[pl.ds(0, _L)]
    for i in range(1, _NS):
        tv = tv + all_v[pl.ds(i * _L, _L)]
    tot = tv[0]
    for l in range(1, _L):
        tot = tot + tv[l]

    tv = jnp.full((_L,), tot, jnp.float32)
    y = 0.5 * (1.0 + tv)
    for _ in range(24):
        y = 0.5 * (y + tv / y)
    r = 1.0 / y

    for b in range(nb):
        g_v[pl.ds(b * _L, _L)] = g_v[pl.ds(b * _L, _L)] * r
    pltpu.sync_copy(g_v, out_hbm.at[pl.ds(c * n + base, rows)])


def kernel(coords, features, len_batch):
    b, n, d = features.shape
    mesh = plsc.VectorSubcoreMesh(
        core_axis_name="c", subcore_axis_name="s", num_cores=_NC, num_subcores=_NS
    )
    rows = n // _NS
    run = pl.kernel(
        functools.partial(_sc_body, n, d),
        out_type=jax.ShapeDtypeStruct((b * n,), features.dtype),
        mesh=mesh,
        scratch_types=[
            pltpu.VMEM((rows * d,), jnp.float32),
            pltpu.VMEM((d,), jnp.float32),
            pltpu.VMEM((d,), jnp.float32),
            pltpu.VMEM((rows,), jnp.float32),
            pltpu.VMEM((_L,), jnp.float32),
            pltpu.VMEM((_NS * _L,), jnp.float32),
            pltpu.VMEM_SHARED((_NS * _L,), jnp.float32),
            pltpu.SemaphoreType.DMA,
        ] + [pltpu.VMEM((4 * _L,), jnp.float32) for _ in range(_L)],
    )
    return run(features.reshape(b * n * d))
